# baseline (speedup 1.0000x reference)
import functools

import jax
import jax.numpy as jnp
from jax import lax
from jax.experimental import pallas as pl
from jax.experimental.pallas import tpu as pltpu
from jax.experimental.pallas import tpu_sc as plsc

NC = 2
NS = 16
LN = 16
G = 128
SB = 3
QC = 16


def _sc_body(sb_per_sub, rows_per_sub, zrows,
             ebuf_hbm,
             e0q0, e0q1, e0q2, e0q3,
             o1q0, o1q1, o1q2, o1q3,
             o2q0, o2q1, o2q2, o2q3,
             o3q0, o3q1, o3q2, o3q3,
             buf0, buf1, eb_a, gbuf_a, sem_ga, sem_sa,
             eb_b, gbuf_b, sem_gb, sem_sb, sem_d):
    c = lax.axis_index("c")
    s = lax.axis_index("s")
    row0 = s * rows_per_sub
    base = s * sb_per_sub
    nz = rows_per_sub // zrows

    def load_edges(t_sb, eb):
        pltpu.sync_copy(ebuf_hbm.at[pl.ds(t_sb * SB, SB)], eb)

    def issue_gathers(table, eb, gb, sem):
        for j in range(SB):
            pltpu.async_copy(table.at[eb.at[j, 0]],
                             gb.at[pl.ds(j * G, G)], sem)

    def wait_gathers(gb, sem):
        for j in range(SB):
            pltpu.make_async_copy(e0q0.at[pl.ds(0, G)],
                                  gb.at[pl.ds(j * G, G)], sem).wait()

    def issue_scatters(accb, gb, eb, sem):
        for j in range(SB):
            pltpu.async_copy(gb.at[pl.ds(j * G, G)],
                             accb.at[eb.at[j, 1]], sem, add=True)

    def wait_scatters(accb, gb, sem):
        for j in range(SB):
            pltpu.make_async_copy(gb.at[pl.ds(j * G, G)],
                                  accb.at[pl.ds(0, G)], sem).wait()

    def compute_scale(gb, eb):
        for j in range(SB):
            @pl.loop(0, G // LN)
            def _(q):
                qb = q * LN
                vv = plsc.bitcast(eb[j, 2, pl.ds(qb, LN)], jnp.float32)
                for i in range(LN):
                    e = j * G + qb + i
                    gb[e, pl.ds(0, LN)] = gb[e, pl.ds(0, LN)] * vv[i]

    def zero_acc(accb):
        @pl.loop(0, zrows)
        def _(r):
            gbuf_a[r, pl.ds(0, LN)] = jnp.zeros((LN,), jnp.float32)

        zcps = [
            pltpu.async_copy(gbuf_a.at[pl.ds(0, zrows)],
                             accb.at[pl.ds(row0 + k * zrows, zrows)],
                             sem_ga)
            for k in range(nz)
        ]
        for cp in zcps:
            cp.wait()

    def drain_issue(accb, out_hbm):
        for k in range(nz):
            pltpu.async_copy(accb.at[pl.ds(row0 + k * zrows, zrows)],
                             out_hbm.at[pl.ds(row0 + k * zrows, zrows)],
                             sem_d)

    def drain_wait(out_hbm):
        for k in range(nz):
            pltpu.make_async_copy(
                buf0.at[pl.ds(0, zrows)],
                out_hbm.at[pl.ds(row0 + k * zrows, zrows)], sem_d).wait()

    def edge_pipeline(table, accb):
        load_edges(base, eb_a)
        issue_gathers(table, eb_a, gbuf_a, sem_ga)

        @pl.loop(0, sb_per_sub // 2)
        def _(t2):
            t = base + 2 * t2
            wait_gathers(gbuf_a, sem_ga)
            compute_scale(gbuf_a, eb_a)

            @pl.when(t2 > 0)
            def _():
                wait_scatters(accb, gbuf_b, sem_sb)

            load_edges(t + 1, eb_b)
            issue_gathers(table, eb_b, gbuf_b, sem_gb)
            issue_scatters(accb, gbuf_a, eb_a, sem_sa)

            wait_gathers(gbuf_b, sem_gb)
            compute_scale(gbuf_b, eb_b)
            wait_scatters(accb, gbuf_a, sem_sa)
            load_edges(t + 2, eb_a)
            issue_gathers(table, eb_a, gbuf_a, sem_ga)
            issue_scatters(accb, gbuf_b, eb_b, sem_sb)

        wait_gathers(gbuf_a, sem_ga)
        wait_scatters(accb, gbuf_b, sem_sb)

    def run_quarter(tq, out1, out2, out3):
        pltpu.sync_copy(tq.at[pl.ds(row0, rows_per_sub)],
                        buf0.at[pl.ds(row0, rows_per_sub)])
        zero_acc(buf1)
        plsc.subcore_barrier()

        edge_pipeline(buf0, buf1)
        plsc.subcore_barrier()

        drain_issue(buf1, out1)
        zero_acc(buf0)
        plsc.subcore_barrier()

        edge_pipeline(buf1, buf0)
        plsc.subcore_barrier()

        drain_wait(out1)
        drain_issue(buf0, out2)
        zero_acc(buf1)
        plsc.subcore_barrier()

        edge_pipeline(buf0, buf1)
        plsc.subcore_barrier()

        drain_wait(out2)
        drain_issue(buf1, out3)
        drain_wait(out3)
        plsc.subcore_barrier()

    @pl.when(c == 0)
    def _():
        run_quarter(e0q0, o1q0, o2q0, o3q0)
        run_quarter(e0q1, o1q1, o2q1, o3q1)

    @pl.when(c == 1)
    def _():
        run_quarter(e0q2, o1q2, o2q2, o3q2)
        run_quarter(e0q3, o1q3, o2q3, o3q3)


def _mean_body(a0, a1, a2, a3, b0, b1, b2, b3,
               c0, c1, c2, c3, d0, d1, d2, d3, o):
    o[:, pl.ds(0 * QC, QC)] = (a0[...] + a1[...] + a2[...] + a3[...]) * 0.25
    o[:, pl.ds(1 * QC, QC)] = (b0[...] + b1[...] + b2[...] + b3[...]) * 0.25
    o[:, pl.ds(2 * QC, QC)] = (c0[...] + c1[...] + c2[...] + c3[...]) * 0.25
    o[:, pl.ds(3 * QC, QC)] = (d0[...] + d1[...] + d2[...] + d3[...]) * 0.25


def kernel(edge_index, edge_vals, user_emb, item_emb):
    nu = user_emb.shape[0]
    ni = item_emb.shape[0]
    n_total = nu + ni
    e_edges = edge_vals.shape[0]

    src = edge_index[0].astype(jnp.int32)
    dst = edge_index[1].astype(jnp.int32)
    val = edge_vals.astype(jnp.float32)

    unit = G * SB * NS
    sb_per_sub = (e_edges + unit - 1) // unit
    sb_per_sub += sb_per_sub % 2
    e_pad = sb_per_sub * unit + G * SB
    pad = e_pad - e_edges
    if pad:
        spread = (jnp.arange(pad, dtype=jnp.int32) * 8) % n_total
        src = jnp.concatenate([src, spread])
        dst = jnp.concatenate([dst, spread])
        val = jnp.concatenate([val, jnp.zeros((pad,), jnp.float32)])
    epack = jnp.stack(
        [src.reshape(-1, G), dst.reshape(-1, G),
         lax.bitcast_convert_type(val, jnp.int32).reshape(-1, G)], axis=1)

    n_pad = ((n_total + NS * 8 - 1) // (NS * 8)) * (NS * 8)
    ego = jnp.concatenate([user_emb, item_emb], axis=0)
    if n_pad != n_total:
        ego = jnp.concatenate(
            [ego, jnp.zeros((n_pad - n_total, 4 * QC), jnp.float32)])
    e0q = [ego[:, q * QC:(q + 1) * QC] for q in range(4)]

    rows_per_sub = n_pad // NS
    zrows = 184
    while rows_per_sub % zrows or zrows % 8 or zrows > SB * G:
        zrows -= 8

    quarter_t = jax.ShapeDtypeStruct((n_pad, QC), jnp.float32)
    mesh = plsc.VectorSubcoreMesh(core_axis_name="c", subcore_axis_name="s")
    dbuf_types = [
        pltpu.VMEM((SB, 3, G), jnp.int32),
        pltpu.VMEM((SB * G, QC), jnp.float32),
        pltpu.SemaphoreType.DMA,
        pltpu.SemaphoreType.DMA,
    ]
    sc_call = pl.kernel(
        functools.partial(_sc_body, sb_per_sub, rows_per_sub, zrows),
        out_type=[quarter_t] * 12,
        mesh=mesh,
        scratch_types=[
            pltpu.VMEM_SHARED((n_pad, QC), jnp.float32),
            pltpu.VMEM_SHARED((n_pad, QC), jnp.float32),
        ] + dbuf_types + dbuf_types + [pltpu.SemaphoreType.DMA],
        compiler_params=pltpu.CompilerParams(use_tc_tiling_on_sc=False,
                                             needs_layout_passes=False),
    )
    outs = sc_call(epack, *e0q)

    br = 2048
    while n_pad % br or br % 8:
        br -= 8
    mean = pl.pallas_call(
        _mean_body,
        grid=(n_pad // br,),
        in_specs=[pl.BlockSpec((br, QC), lambda i: (i, 0))] * 16,
        out_specs=pl.BlockSpec((br, 4 * QC), lambda i: (i, 0)),
        out_shape=jax.ShapeDtypeStruct((n_pad, 4 * QC), jnp.float32),
    )(e0q[0], outs[0], outs[4], outs[8],
      e0q[1], outs[1], outs[5], outs[9],
      e0q[2], outs[2], outs[6], outs[10],
      e0q[3], outs[3], outs[7], outs[11])

    return mean[:nu], mean[nu:n_total]


# speedup vs baseline: 1.0043x; 1.0043x over previous
import functools

import jax
import jax.numpy as jnp
from jax import lax
from jax.experimental import pallas as pl
from jax.experimental.pallas import tpu as pltpu
from jax.experimental.pallas import tpu_sc as plsc

NC = 2
NS = 16
LN = 16
G = 128
SB = 3
QC = 16


def _sc_body(sb_per_sub, rows_per_sub, zrows,
             ebuf_hbm,
             e0q0, e0q1, e0q2, e0q3,
             o1q0, o1q1, o1q2, o1q3,
             o2q0, o2q1, o2q2, o2q3,
             o3q0, o3q1, o3q2, o3q3,
             buf0, buf1, eb_a, gbuf_a, sem_ga, sem_sa,
             eb_b, gbuf_b, sem_gb, sem_sb, sem_d):
    c = lax.axis_index("c")
    s = lax.axis_index("s")
    row0 = s * rows_per_sub
    base = s * sb_per_sub
    nz = rows_per_sub // zrows

    def load_edges(t_sb, eb):
        pltpu.sync_copy(ebuf_hbm.at[pl.ds(t_sb * SB, SB)], eb)

    def issue_gathers(table, eb, gb, sem):
        for j in range(SB):
            pltpu.async_copy(table.at[eb.at[j, 0]],
                             gb.at[pl.ds(j * G, G)], sem)

    def wait_gathers(gb, sem):
        pltpu.make_async_copy(e0q0.at[pl.ds(0, SB * G)], gb, sem).wait()

    def issue_scatters(accb, gb, eb, sem):
        for j in range(SB):
            pltpu.async_copy(gb.at[pl.ds(j * G, G)],
                             accb.at[eb.at[j, 1]], sem, add=True)

    def wait_scatters(accb, gb, sem):
        pltpu.make_async_copy(gb, accb.at[pl.ds(0, SB * G)], sem).wait()

    def compute_scale(gb, eb):
        for j in range(SB):
            @pl.loop(0, G // LN)
            def _(q):
                qb = q * LN
                vv = plsc.bitcast(eb[j, 2, pl.ds(qb, LN)], jnp.float32)
                for i in range(LN):
                    e = j * G + qb + i
                    gb[e, pl.ds(0, LN)] = gb[e, pl.ds(0, LN)] * vv[i]

    def zero_acc(accb):
        @pl.loop(0, zrows)
        def _(r):
            gbuf_a[r, pl.ds(0, LN)] = jnp.zeros((LN,), jnp.float32)

        zcps = [
            pltpu.async_copy(gbuf_a.at[pl.ds(0, zrows)],
                             accb.at[pl.ds(row0 + k * zrows, zrows)],
                             sem_ga)
            for k in range(nz)
        ]
        for cp in zcps:
            cp.wait()

    def drain_issue(accb, out_hbm):
        for k in range(nz):
            pltpu.async_copy(accb.at[pl.ds(row0 + k * zrows, zrows)],
                             out_hbm.at[pl.ds(row0 + k * zrows, zrows)],
                             sem_d)

    def drain_wait(out_hbm):
        for k in range(nz):
            pltpu.make_async_copy(
                buf0.at[pl.ds(0, zrows)],
                out_hbm.at[pl.ds(row0 + k * zrows, zrows)], sem_d).wait()

    def edge_pipeline(table, accb):
        load_edges(base, eb_a)
        issue_gathers(table, eb_a, gbuf_a, sem_ga)

        @pl.loop(0, sb_per_sub // 2)
        def _(t2):
            t = base + 2 * t2
            wait_gathers(gbuf_a, sem_ga)
            compute_scale(gbuf_a, eb_a)

            @pl.when(t2 > 0)
            def _():
                wait_scatters(accb, gbuf_b, sem_sb)

            load_edges(t + 1, eb_b)
            issue_gathers(table, eb_b, gbuf_b, sem_gb)
            issue_scatters(accb, gbuf_a, eb_a, sem_sa)

            wait_gathers(gbuf_b, sem_gb)
            compute_scale(gbuf_b, eb_b)
            wait_scatters(accb, gbuf_a, sem_sa)
            load_edges(t + 2, eb_a)
            issue_gathers(table, eb_a, gbuf_a, sem_ga)
            issue_scatters(accb, gbuf_b, eb_b, sem_sb)

        wait_gathers(gbuf_a, sem_ga)
        wait_scatters(accb, gbuf_b, sem_sb)

    def run_quarter(tq, out1, out2, out3):
        pltpu.sync_copy(tq.at[pl.ds(row0, rows_per_sub)],
                        buf0.at[pl.ds(row0, rows_per_sub)])
        zero_acc(buf1)
        plsc.subcore_barrier()

        edge_pipeline(buf0, buf1)
        plsc.subcore_barrier()

        drain_issue(buf1, out1)
        zero_acc(buf0)
        plsc.subcore_barrier()

        edge_pipeline(buf1, buf0)
        plsc.subcore_barrier()

        drain_wait(out1)
        drain_issue(buf0, out2)
        zero_acc(buf1)
        plsc.subcore_barrier()

        edge_pipeline(buf0, buf1)
        plsc.subcore_barrier()

        drain_wait(out2)
        drain_issue(buf1, out3)
        drain_wait(out3)
        plsc.subcore_barrier()

    @pl.when(c == 0)
    def _():
        run_quarter(e0q0, o1q0, o2q0, o3q0)
        run_quarter(e0q1, o1q1, o2q1, o3q1)

    @pl.when(c == 1)
    def _():
        run_quarter(e0q2, o1q2, o2q2, o3q2)
        run_quarter(e0q3, o1q3, o2q3, o3q3)


def _mean_body(a0, a1, a2, a3, b0, b1, b2, b3,
               c0, c1, c2, c3, d0, d1, d2, d3, o):
    o[:, pl.ds(0 * QC, QC)] = (a0[...] + a1[...] + a2[...] + a3[...]) * 0.25
    o[:, pl.ds(1 * QC, QC)] = (b0[...] + b1[...] + b2[...] + b3[...]) * 0.25
    o[:, pl.ds(2 * QC, QC)] = (c0[...] + c1[...] + c2[...] + c3[...]) * 0.25
    o[:, pl.ds(3 * QC, QC)] = (d0[...] + d1[...] + d2[...] + d3[...]) * 0.25


def kernel(edge_index, edge_vals, user_emb, item_emb):
    nu = user_emb.shape[0]
    ni = item_emb.shape[0]
    n_total = nu + ni
    e_edges = edge_vals.shape[0]

    src = edge_index[0].astype(jnp.int32)
    dst = edge_index[1].astype(jnp.int32)
    val = edge_vals.astype(jnp.float32)

    unit = G * SB * NS
    sb_per_sub = (e_edges + unit - 1) // unit
    sb_per_sub += sb_per_sub % 2
    e_pad = sb_per_sub * unit + G * SB
    pad = e_pad - e_edges
    if pad:
        spread = (jnp.arange(pad, dtype=jnp.int32) * 8) % n_total
        src = jnp.concatenate([src, spread])
        dst = jnp.concatenate([dst, spread])
        val = jnp.concatenate([val, jnp.zeros((pad,), jnp.float32)])
    epack = jnp.stack(
        [src.reshape(-1, G), dst.reshape(-1, G),
         lax.bitcast_convert_type(val, jnp.int32).reshape(-1, G)], axis=1)

    n_pad = ((n_total + NS * 8 - 1) // (NS * 8)) * (NS * 8)
    ego = jnp.concatenate([user_emb, item_emb], axis=0)
    if n_pad != n_total:
        ego = jnp.concatenate(
            [ego, jnp.zeros((n_pad - n_total, 4 * QC), jnp.float32)])
    e0q = [ego[:, q * QC:(q + 1) * QC] for q in range(4)]

    rows_per_sub = n_pad // NS
    zrows = 184
    while rows_per_sub % zrows or zrows % 8 or zrows > SB * G:
        zrows -= 8

    quarter_t = jax.ShapeDtypeStruct((n_pad, QC), jnp.float32)
    mesh = plsc.VectorSubcoreMesh(core_axis_name="c", subcore_axis_name="s")
    dbuf_types = [
        pltpu.VMEM((SB, 3, G), jnp.int32),
        pltpu.VMEM((SB * G, QC), jnp.float32),
        pltpu.SemaphoreType.DMA,
        pltpu.SemaphoreType.DMA,
    ]
    sc_call = pl.kernel(
        functools.partial(_sc_body, sb_per_sub, rows_per_sub, zrows),
        out_type=[quarter_t] * 12,
        mesh=mesh,
        scratch_types=[
            pltpu.VMEM_SHARED((n_pad, QC), jnp.float32),
            pltpu.VMEM_SHARED((n_pad, QC), jnp.float32),
        ] + dbuf_types + dbuf_types + [pltpu.SemaphoreType.DMA],
        compiler_params=pltpu.CompilerParams(use_tc_tiling_on_sc=False,
                                             needs_layout_passes=False),
    )
    outs = sc_call(epack, *e0q)

    br = 2048
    while n_pad % br or br % 8:
        br -= 8
    mean = pl.pallas_call(
        _mean_body,
        grid=(n_pad // br,),
        in_specs=[pl.BlockSpec((br, QC), lambda i: (i, 0))] * 16,
        out_specs=pl.BlockSpec((br, 4 * QC), lambda i: (i, 0)),
        out_shape=jax.ShapeDtypeStruct((n_pad, 4 * QC), jnp.float32),
    )(e0q[0], outs[0], outs[4], outs[8],
      e0q[1], outs[1], outs[5], outs[9],
      e0q[2], outs[2], outs[6], outs[10],
      e0q[3], outs[3], outs[7], outs[11])

    return mean[:nu], mean[nu:n_total]
